# Initial kernel scaffold; baseline (speedup 1.0000x reference)
import jax
import jax.numpy as jnp
from jax.experimental import pallas as pl


def kernel(x, token_table, pos_table):
    raise NotImplementedError("write your pallas kernel here")


# speedup vs baseline: 1.4351x; 1.4351x over previous
import functools

import jax
import jax.numpy as jnp
from jax import lax
from jax.experimental import pallas as pl
from jax.experimental.pallas import tpu as pltpu
from jax.experimental.pallas import tpu_sc as plsc

NC = 2
NS = 16
NW = NC * NS

LANES = 16

BATCH = 4096
MAXLEN = 200
EMBED = 32

IDX_PER_CALL = 100
ROWS_PER_CHUNK_B = 8
ROWS_PER_CHUNK = ROWS_PER_CHUNK_B * MAXLEN
CALLS_PER_CHUNK = ROWS_PER_CHUNK // IDX_PER_CALL
BATCH_PER_W = BATCH // NW
CHUNKS = BATCH_PER_W // ROWS_PER_CHUNK_B
FLAT_PER_W = BATCH_PER_W * MAXLEN
CALLS_PER_W = FLAT_PER_W // IDX_PER_CALL


def _body(x_hbm, tok_hbm, pos_hbm, out_hbm, idx_v, rows_v, pos_v, sem):
    wid = lax.axis_index("s") * NC + lax.axis_index("c")

    pltpu.sync_copy(pos_hbm, pos_v)
    pltpu.sync_copy(x_hbm.at[pl.ds(wid * CALLS_PER_W, CALLS_PER_W)], idx_v)

    def chunk_body(c, carry):
        descs = []
        for j in range(CALLS_PER_CHUNK):
            d = pltpu.async_copy(
                tok_hbm.at[idx_v.at[c * CALLS_PER_CHUNK + j]],
                rows_v.at[pl.ds(j * IDX_PER_CALL, IDX_PER_CALL)],
                sem,
            )
            descs.append(d)
        for d in descs:
            d.wait()

        def add_body(l, acc):
            p0 = pos_v[l, pl.ds(0, LANES)]
            p1 = pos_v[l, pl.ds(LANES, LANES)]
            for r in range(ROWS_PER_CHUNK_B):
                row = r * MAXLEN + l
                rows_v[row, pl.ds(0, LANES)] = rows_v[row, pl.ds(0, LANES)] + p0
                rows_v[row, pl.ds(LANES, LANES)] = (
                    rows_v[row, pl.ds(LANES, LANES)] + p1
                )
            return acc

        lax.fori_loop(0, MAXLEN, add_body, 0)

        pltpu.sync_copy(
            rows_v,
            out_hbm.at[pl.ds(wid * FLAT_PER_W + c * ROWS_PER_CHUNK, ROWS_PER_CHUNK)],
        )
        return carry

    lax.fori_loop(0, CHUNKS, chunk_body, 0)


@jax.jit
def kernel(x, token_table, pos_table):
    mesh = plsc.VectorSubcoreMesh(
        core_axis_name="c", subcore_axis_name="s", num_cores=NC, num_subcores=NS
    )
    x2 = x.reshape(-1, IDX_PER_CALL)
    out = pl.kernel(
        _body,
        out_type=jax.ShapeDtypeStruct((BATCH * MAXLEN, EMBED), jnp.float32),
        mesh=mesh,
        scratch_types=[
            pltpu.VMEM((CALLS_PER_W, IDX_PER_CALL), jnp.int32),
            pltpu.VMEM((ROWS_PER_CHUNK, EMBED), jnp.float32),
            pltpu.VMEM((MAXLEN, EMBED), jnp.float32),
            pltpu.SemaphoreType.DMA,
        ],
        compiler_params=pltpu.CompilerParams(use_tc_tiling_on_sc=False),
    )(x2, token_table, pos_table)
    return out.reshape(BATCH, MAXLEN, EMBED)
